# baseline (speedup 1.0000x reference)
import jax
import jax.numpy as jnp
from jax.experimental import pallas as pl
from jax.experimental.pallas import tpu as pltpu


def _gat(x, edge_index, W, att_src, att_dst, bias, heads, C, concat):
    n = x.shape[0]
    src = edge_index[0]
    dst = edge_index[1]
    loop = jnp.arange(n, dtype=src.dtype)
    src = jnp.concatenate([src, loop])
    dst = jnp.concatenate([dst, loop])
    h = (x @ W).reshape(n, heads, C)
    a_src = (h * att_src[None, :, :]).sum(-1)
    a_dst = (h * att_dst[None, :, :]).sum(-1)
    alpha = a_src[src] + a_dst[dst]
    alpha = jax.nn.leaky_relu(alpha, 0.2)
    amax = jax.ops.segment_max(alpha, dst, num_segments=n)
    alpha = jnp.exp(alpha - amax[dst])
    denom = jax.ops.segment_sum(alpha, dst, num_segments=n)
    alpha = alpha / (denom[dst] + 1e-16)
    msg = h[src] * alpha[:, :, None]
    out = jax.ops.segment_sum(msg, dst, num_segments=n)
    if concat:
        out = out.reshape(n, heads * C)
    else:
        out = out.mean(axis=1)
    return out + bias


def _logsoftmax_body(x_ref, o_ref):
    x = x_ref[...]
    m = jnp.max(x, axis=1, keepdims=True)
    e = jnp.exp(x - m)
    o_ref[...] = x - m - jnp.log(jnp.sum(e, axis=1, keepdims=True))


def kernel(x, edge_index, W1, att_src1, att_dst1, b1, W2, att_src2, att_dst2, b2):
    h = _gat(x, edge_index, W1, att_src1, att_dst1, b1, heads=2, C=16, concat=True)
    h = jax.nn.relu(h)
    h = _gat(h, edge_index, W2, att_src2, att_dst2, b2, heads=2, C=7, concat=False)
    blk = 1000
    return pl.pallas_call(
        _logsoftmax_body,
        grid=(h.shape[0] // blk,),
        in_specs=[pl.BlockSpec((blk, h.shape[1]), lambda i: (i, 0))],
        out_specs=pl.BlockSpec((blk, h.shape[1]), lambda i: (i, 0)),
        out_shape=jax.ShapeDtypeStruct(h.shape, h.dtype),
    )(h)


# speedup vs baseline: 11.8549x; 11.8549x over previous
import functools

import jax
import jax.numpy as jnp
from jax import lax
from jax.experimental import pallas as pl
from jax.experimental.pallas import tpu as pltpu
from jax.experimental.pallas import tpu_sc as plsc

N = 100000
CHK = 16
HALFP = 50176
NPAD = 2 * HALFP
HALF4 = HALFP // 4
NPAD4 = 2 * HALF4
TPQ = HALF4 // 16
BLK = 5000
NBLK = N // BLK
EBLK = 2000
NEBLK = N // EBLK


def _splat(col, width):
    return jnp.broadcast_to(col, (col.shape[0], width))


def _tc1_body(x_ref, w_ref, as_ref, ad_ref, ta_ref, tb_ref, ada_ref, adb_ref,
              asv_ref, adv_ref, pm_ref):
    xb = x_ref[...]
    W = w_ref[...]
    h = jnp.dot(xb, W, preferred_element_type=jnp.float32)
    a_s = as_ref[...]
    a_d = ad_ref[...]
    asv = jnp.concatenate(
        [jnp.sum(h[:, :16] * a_s[0][None, :], axis=1, keepdims=True),
         jnp.sum(h[:, 16:] * a_s[1][None, :], axis=1, keepdims=True)], axis=1)
    adv = jnp.concatenate(
        [jnp.sum(h[:, :16] * a_d[0][None, :], axis=1, keepdims=True),
         jnp.sum(h[:, 16:] * a_d[1][None, :], axis=1, keepdims=True)], axis=1)
    z96 = jnp.zeros((xb.shape[0], 96), jnp.float32)
    z112 = jnp.zeros((xb.shape[0], 112), jnp.float32)
    ta_ref[...] = jnp.concatenate(
        [h[:, :16], _splat(asv[:, 0:1], 16), z96], axis=1)
    tb_ref[...] = jnp.concatenate(
        [h[:, 16:], _splat(asv[:, 1:2], 16), z96], axis=1)
    ada_ref[...] = jnp.concatenate([_splat(adv[:, 0:1], 16), z112], axis=1)
    adb_ref[...] = jnp.concatenate([_splat(adv[:, 1:2], 16), z112], axis=1)
    asv_ref[...] = asv
    adv_ref[...] = adv
    m = jnp.concatenate([jnp.max(asv, axis=0), jnp.max(adv, axis=0),
                         jnp.full((124,), -1e30, jnp.float32)])
    pm_ref[...] = m[None, None, :]


def _kred_body(pm_ref, k_ref):
    m = jnp.max(pm_ref[...][:, 0, :], axis=0)
    s = m[0:2] + m[2:4]
    k = jnp.maximum(s, 0.2 * s)
    k_ref[...] = jnp.broadcast_to(k[:, None], (2, 16))


def _expand(w, width):
    return jnp.concatenate(
        [jnp.broadcast_to(w[:, 0:1], (w.shape[0], width)),
         jnp.broadcast_to(w[:, 1:2], (w.shape[0], width))], axis=1)


def _epi1_body(acca_ref, accb_ref, ta_ref, tb_ref, asv_ref, adv_ref, k_ref,
               b1_ref, w2_ref, as2_ref, ad2_ref, t2_ref, ad2t_ref, asv2_ref,
               adv2_ref, pm2_ref):
    acca = acca_ref[...]
    accb = accb_ref[...]
    h = jnp.concatenate([ta_ref[...][:, :16], tb_ref[...][:, :16]], axis=1)
    asv = asv_ref[...]
    adv = adv_ref[...]
    k = k_ref[...]
    s = asv + adv
    lr = jnp.maximum(s, 0.2 * s)
    wself = jnp.exp(lr - k[:, 0][None, :])
    num = jnp.concatenate([acca[:, :16], accb[:, :16]], axis=1) \
        + h * _expand(wself, 16)
    den = jnp.concatenate([acca[:, 16:17], accb[:, 16:17]], axis=1) + wself
    out = num / _expand(den, 16) + b1_ref[...][None, :]
    h2 = jnp.maximum(out, 0.0)
    g = jnp.dot(h2, w2_ref[...], preferred_element_type=jnp.float32)
    a_s2 = as2_ref[...]
    a_d2 = ad2_ref[...]
    asv2 = jnp.concatenate(
        [jnp.sum(g[:, :7] * a_s2[0][None, :], axis=1, keepdims=True),
         jnp.sum(g[:, 7:] * a_s2[1][None, :], axis=1, keepdims=True)], axis=1)
    adv2 = jnp.concatenate(
        [jnp.sum(g[:, :7] * a_d2[0][None, :], axis=1, keepdims=True),
         jnp.sum(g[:, 7:] * a_d2[1][None, :], axis=1, keepdims=True)], axis=1)
    ci = lax.broadcasted_iota(jnp.int32, (g.shape[0], 128), 1)
    gpad = jnp.concatenate([g, jnp.zeros((g.shape[0], 114), jnp.float32)],
                           axis=1)
    as0 = _splat(asv2[:, 0:1], 128)
    as1 = _splat(asv2[:, 1:2], 128)
    t2_ref[...] = jnp.where(
        ci < 14, gpad,
        jnp.where(ci < 16, 0.0,
                  jnp.where(ci < 23, as0, jnp.where(ci < 30, as1, 0.0))))
    ad0 = _splat(adv2[:, 0:1], 128)
    ad1 = _splat(adv2[:, 1:2], 128)
    ad2t_ref[...] = jnp.where(ci < 7, ad0, jnp.where(ci < 14, ad1, 0.0))
    asv2_ref[...] = asv2
    adv2_ref[...] = adv2
    m = jnp.concatenate([jnp.max(asv2, axis=0), jnp.max(adv2, axis=0),
                         jnp.full((124,), -1e30, jnp.float32)])
    pm2_ref[...] = m[None, None, :]


def _epi2_body(acc_ref, t2_ref, asv2_ref, adv2_ref, k_ref, b2_ref, o_ref):
    acc = acc_ref[...]
    g = t2_ref[...][:, 0:14]
    asv2 = asv2_ref[...]
    adv2 = adv2_ref[...]
    k = k_ref[...]
    s = asv2 + adv2
    lr = jnp.maximum(s, 0.2 * s)
    wself = jnp.exp(lr - k[:, 0][None, :])
    num = acc[:, 0:14] + g * _expand(wself, 7)
    den = jnp.concatenate([acc[:, 16:17], acc[:, 23:24]], axis=1) + wself
    out = num / _expand(den, 7)
    z = (out[:, :7] + out[:, 7:14]) * 0.5 + b2_ref[...][None, :]
    m = jnp.max(z, axis=1, keepdims=True)
    e = jnp.exp(z - m)
    o_ref[...] = z - m - jnp.log(jnp.sum(e, axis=1, keepdims=True))


def _make_edge_pass(pe, e_real, mixk):
    ch = pe // 16 // CHK
    ep = pe // 16
    mesh = plsc.VectorSubcoreMesh(core_axis_name="c", subcore_axis_name="s")

    scratch = [
        pltpu.VMEM_SHARED((HALF4 + 8, 128), jnp.float32),
        pltpu.VMEM((CHK,), jnp.int32),
        pltpu.VMEM((CHK,), jnp.int32),
        pltpu.VMEM((CHK,), jnp.int32),
        pltpu.VMEM((CHK, 128), jnp.float32),
        pltpu.VMEM((CHK, 128), jnp.float32),
        pltpu.VMEM((CHK, 128), jnp.float32),
        pltpu.VMEM((32 if mixk else 16,), jnp.float32),
        pltpu.SemaphoreType.DMA,
    ]

    @functools.partial(
        pl.kernel,
        out_type=jax.ShapeDtypeStruct((NPAD4, 128), jnp.float32),
        mesh=mesh,
        scratch_types=scratch,
    )
    def f(srcp, dstp, tbl, adt, kk, zr, out_hbm,
          accum, srcv, dstv, idxv, G, AD, MSG, KV, sem):
        c = lax.axis_index("c")
        s = lax.axis_index("s")
        base = c * HALFP
        r0 = s * TPQ
        lanes = lax.iota(jnp.int32, 16)

        pltpu.sync_copy(kk, KV)
        if mixk:
            kvec = jnp.where(lanes < 7, KV[pl.ds(0, 16)],
                             jnp.where(lanes < 14, KV[pl.ds(16, 16)], 1e30))
        else:
            kvec = KV[...]
        pltpu.sync_copy(zr, accum.at[pl.ds(r0, TPQ)])
        plsc.subcore_barrier()

        zero16 = jnp.zeros((16,), jnp.float32)

        def chunk(j, carry):
            off = s * ep + j * CHK
            pltpu.sync_copy(srcp.at[pl.ds(off, CHK)], srcv)
            pltpu.sync_copy(dstp.at[pl.ds(off, CHK)], dstv)
            cp1 = pltpu.async_copy(tbl.at[srcv], G, sem)
            cp2 = pltpu.async_copy(adt.at[dstv], AD, sem)
            cp1.wait()
            cp2.wait()
            for g_ in range(CHK // 16):
                sl = pl.ds(g_ * 16, 16)
                dv = dstv[sl]
                gi = off + g_ * 16 + lanes
                dl = dv - base
                valid = (dl >= 0) & (dl < HALFP) & (gi < e_real)
                dl = jnp.where(valid, dl, 0)
                idxv[sl] = jnp.where(valid, dl >> 2, HALF4)
                qv = dl & 3
                for l in range(16):
                    e = g_ * 16 + l
                    w = jnp.exp(jnp.maximum(
                        G[e, pl.ds(16, 16)] + AD[e, pl.ds(0, 16)],
                        0.2 * (G[e, pl.ds(16, 16)] + AD[e, pl.ds(0, 16)]))
                        - kvec)
                    mrow = G[e, pl.ds(0, 16)] * w
                    q = qv[l]
                    for grp in range(8):
                        part = mrow if grp % 2 == 0 else w
                        sel = jnp.full(
                            (16,),
                            jnp.where(q == (grp // 2), 1.0, 0.0).astype(
                                jnp.float32))
                        MSG[e, pl.ds(grp * 16, 16)] = part * sel
            pltpu.sync_copy(MSG, accum.at[idxv], add=True)
            return carry

        lax.fori_loop(0, ch, chunk, 0)
        plsc.subcore_barrier()
        pltpu.sync_copy(accum.at[pl.ds(r0, TPQ)],
                        out_hbm.at[pl.ds(c * HALF4 + r0, TPQ)])

    return f


def kernel(x, edge_index, W1, att_src1, att_dst1, b1, W2, att_src2, att_dst2, b2):
    e_real = edge_index.shape[1]
    pe = ((e_real + 16 * CHK - 1) // (16 * CHK)) * (16 * CHK)
    pad = pe - e_real
    srcp = jnp.concatenate([edge_index[0], jnp.zeros((pad,), jnp.int32)])
    dstp = jnp.concatenate([edge_index[1], jnp.zeros((pad,), jnp.int32)])

    ta, tb, ada, adb, asv1, adv1, pm1 = pl.pallas_call(
        _tc1_body,
        grid=(NBLK,),
        in_specs=[
            pl.BlockSpec((BLK, 3), lambda i: (i, 0)),
            pl.BlockSpec((3, 32), lambda i: (0, 0)),
            pl.BlockSpec((2, 16), lambda i: (0, 0)),
            pl.BlockSpec((2, 16), lambda i: (0, 0)),
        ],
        out_specs=[
            pl.BlockSpec((BLK, 128), lambda i: (i, 0)),
            pl.BlockSpec((BLK, 128), lambda i: (i, 0)),
            pl.BlockSpec((BLK, 128), lambda i: (i, 0)),
            pl.BlockSpec((BLK, 128), lambda i: (i, 0)),
            pl.BlockSpec((BLK, 2), lambda i: (i, 0)),
            pl.BlockSpec((BLK, 2), lambda i: (i, 0)),
            pl.BlockSpec((1, 1, 128), lambda i: (i, 0, 0)),
        ],
        out_shape=[
            jax.ShapeDtypeStruct((N, 128), jnp.float32),
            jax.ShapeDtypeStruct((N, 128), jnp.float32),
            jax.ShapeDtypeStruct((N, 128), jnp.float32),
            jax.ShapeDtypeStruct((N, 128), jnp.float32),
            jax.ShapeDtypeStruct((N, 2), jnp.float32),
            jax.ShapeDtypeStruct((N, 2), jnp.float32),
            jax.ShapeDtypeStruct((NBLK, 1, 128), jnp.float32),
        ],
    )(x, W1, att_src1, att_dst1)

    k1 = pl.pallas_call(
        _kred_body,
        out_shape=jax.ShapeDtypeStruct((2, 16), jnp.float32),
    )(pm1)

    zr = jnp.zeros((TPQ, 128), jnp.float32)
    edge1 = _make_edge_pass(pe, e_real, False)
    acca = edge1(srcp, dstp, ta, ada, k1[0], zr).reshape(NPAD, 32)
    accb = edge1(srcp, dstp, tb, adb, k1[1], zr).reshape(NPAD, 32)

    T2, AD2, asv2, adv2, pm2 = pl.pallas_call(
        _epi1_body,
        grid=(NEBLK,),
        in_specs=[
            pl.BlockSpec((EBLK, 32), lambda i: (i, 0)),
            pl.BlockSpec((EBLK, 32), lambda i: (i, 0)),
            pl.BlockSpec((EBLK, 128), lambda i: (i, 0)),
            pl.BlockSpec((EBLK, 128), lambda i: (i, 0)),
            pl.BlockSpec((EBLK, 2), lambda i: (i, 0)),
            pl.BlockSpec((EBLK, 2), lambda i: (i, 0)),
            pl.BlockSpec((2, 16), lambda i: (0, 0)),
            pl.BlockSpec((32,), lambda i: (0,)),
            pl.BlockSpec((32, 14), lambda i: (0, 0)),
            pl.BlockSpec((2, 7), lambda i: (0, 0)),
            pl.BlockSpec((2, 7), lambda i: (0, 0)),
        ],
        out_specs=[
            pl.BlockSpec((EBLK, 128), lambda i: (i, 0)),
            pl.BlockSpec((EBLK, 128), lambda i: (i, 0)),
            pl.BlockSpec((EBLK, 2), lambda i: (i, 0)),
            pl.BlockSpec((EBLK, 2), lambda i: (i, 0)),
            pl.BlockSpec((1, 1, 128), lambda i: (i, 0, 0)),
        ],
        out_shape=[
            jax.ShapeDtypeStruct((N, 128), jnp.float32),
            jax.ShapeDtypeStruct((N, 128), jnp.float32),
            jax.ShapeDtypeStruct((N, 2), jnp.float32),
            jax.ShapeDtypeStruct((N, 2), jnp.float32),
            jax.ShapeDtypeStruct((NEBLK, 1, 128), jnp.float32),
        ],
    )(acca[:N], accb[:N], ta, tb, asv1, adv1, k1, b1, W2, att_src2, att_dst2)

    k2 = pl.pallas_call(
        _kred_body,
        out_shape=jax.ShapeDtypeStruct((2, 16), jnp.float32),
    )(pm2)

    edge2 = _make_edge_pass(pe, e_real, True)
    acc2 = edge2(srcp, dstp, T2, AD2, k2.reshape(-1), zr).reshape(NPAD, 32)

    out = pl.pallas_call(
        _epi2_body,
        grid=(NBLK,),
        in_specs=[
            pl.BlockSpec((BLK, 32), lambda i: (i, 0)),
            pl.BlockSpec((BLK, 128), lambda i: (i, 0)),
            pl.BlockSpec((BLK, 2), lambda i: (i, 0)),
            pl.BlockSpec((BLK, 2), lambda i: (i, 0)),
            pl.BlockSpec((2, 16), lambda i: (0, 0)),
            pl.BlockSpec((7,), lambda i: (0,)),
        ],
        out_specs=pl.BlockSpec((BLK, 7), lambda i: (i, 0)),
        out_shape=jax.ShapeDtypeStruct((N, 7), jnp.float32),
    )(acc2[:N], T2, asv2, adv2, k2, b2)

    return out
